# baseline (speedup 1.0000x reference)
import functools

import jax
import jax.numpy as jnp
from jax import lax
from jax.experimental import pallas as pl
from jax.experimental.pallas import tpu as pltpu
from jax.experimental.pallas import tpu_sc as plsc

N = 10000
E = 320000
D = 128
NC = 2
NS = 16
NW = NC * NS
EPW = E // NW
K = 125
CH = EPW // K
NP = 2
CH2 = CH // NP
RB = 1000
RB2 = N // NS

_sc_mesh = plsc.VectorSubcoreMesh(core_axis_name="c", subcore_axis_name="s")


@functools.partial(
    pl.kernel,
    out_type=jax.ShapeDtypeStruct((NC * N,), jnp.float32),
    mesh=_sc_mesh,
    scratch_types=[
        pltpu.VMEM((CH2, K), jnp.int32),
        pltpu.VMEM((K,), jnp.float32),
        pltpu.VMEM((RB,), jnp.float32),
        pltpu.VMEM_SHARED((N,), jnp.float32),
    ],
)
def _sc_degree(er_hbm, ones_hbm, zeros1_hbm, deg_hbm, dst_v, ones_v, stage_v,
               acc):
    cid = lax.axis_index("c")
    sid = lax.axis_index("s")
    wid = cid * NS + sid

    @pl.when(sid < N // RB)
    def _():
        rs = pl.ds(sid * RB, RB)
        pltpu.sync_copy(zeros1_hbm.at[rs], stage_v)
        pltpu.sync_copy(stage_v, acc.at[rs])

    pltpu.sync_copy(ones_hbm, ones_v)
    plsc.subcore_barrier()

    for p in range(NP):
        pltpu.sync_copy(er_hbm.at[NP * NW + NP * wid + p], dst_v)

        @pl.loop(0, CH2)
        def _(j):
            pltpu.sync_copy(ones_v, acc.at[dst_v.at[j]], add=True)

    plsc.subcore_barrier()

    @pl.when(sid < N // RB)
    def _():
        rs = pl.ds(sid * RB, RB)
        pltpu.sync_copy(acc.at[rs], stage_v)
        pltpu.sync_copy(stage_v, deg_hbm.at[pl.ds(cid * N + sid * RB, RB)])


@functools.partial(
    pl.kernel,
    out_type=jax.ShapeDtypeStruct((NC, N, D), jnp.float32),
    mesh=_sc_mesh,
    scratch_types=[
        pltpu.VMEM((CH2, K), jnp.int32),
        pltpu.VMEM((CH2, K), jnp.int32),
        pltpu.VMEM((128, D), jnp.float32),
        pltpu.VMEM((128, D), jnp.float32),
        pltpu.SemaphoreType.DMA,
        pltpu.SemaphoreType.DMA,
        pltpu.SemaphoreType.DMA,
        pltpu.SemaphoreType.DMA,
        pltpu.VMEM_SHARED((N, D), jnp.float32),
    ],
)
def _sc_scatter(er_hbm, xs_hbm, out_hbm,
                src_v, dst_v, rows0, rows1, sg0, sg1, ss0, ss1, acc):
    cid = lax.axis_index("c")
    sid = lax.axis_index("s")
    wid = cid * NS + sid

    @pl.when(sid < N // RB)
    def _():
        rs = pl.ds(sid * RB, RB)

        @pl.when(cid == 0)
        def _():
            pltpu.sync_copy(xs_hbm.at[rs], acc.at[rs])

        @pl.when(cid == 1)
        def _():
            @pl.loop(0, 128)
            def _(r):
                for c in range(D // 16):
                    rows0[r, pl.ds(c * 16, 16)] = jnp.zeros((16,), jnp.float32)

            for i in range(7):
                pltpu.sync_copy(
                    rows0, acc.at[pl.ds(sid * RB + i * 128, 128)])
            pltpu.sync_copy(rows0.at[pl.ds(0, RB - 7 * 128)],
                            acc.at[pl.ds(sid * RB + 7 * 128, RB - 7 * 128)])

    plsc.subcore_barrier()

    rows = (rows0.at[pl.ds(0, K)], rows1.at[pl.ds(0, K)])
    sg = (sg0, sg1)
    ss = (ss0, ss1)

    for p in range(NP):
        pltpu.sync_copy(er_hbm.at[NP * wid + p], src_v)
        pltpu.sync_copy(er_hbm.at[NP * NW + NP * wid + p], dst_v)
        pltpu.async_copy(xs_hbm.at[src_v.at[0]], rows[0], sg[0])

        @pl.loop(0, CH2)
        def _(j):
            for b in range(2):
                @pl.when(j % 2 == b)
                def _():
                    pltpu.make_async_copy(
                        xs_hbm.at[src_v.at[j]], rows[b], sg[b]).wait()
                    pltpu.async_copy(
                        rows[b], acc.at[dst_v.at[j]], ss[b], add=True)

                    @pl.when(j > 0)
                    def _():
                        pltpu.make_async_copy(
                            rows[1 - b], acc.at[dst_v.at[j - 1]],
                            ss[1 - b]).wait()

                    @pl.when(j < CH2 - 1)
                    def _():
                        pltpu.async_copy(
                            xs_hbm.at[src_v.at[j + 1]], rows[1 - b], sg[1 - b])

        bl = (CH2 - 1) % 2
        pltpu.make_async_copy(
            rows[bl], acc.at[dst_v.at[CH2 - 1]], ss[bl]).wait()

    plsc.subcore_barrier()

    @pl.when(sid < N // RB)
    def _():
        rs = pl.ds(sid * RB, RB)
        pltpu.sync_copy(acc.at[rs], out_hbm.at[cid, rs])


R = 1000
G = N // R


def _tc_mm_body(z_ref, w_ref, x_ref):
    x_ref[...] = jnp.dot(z_ref[...], w_ref[...],
                         preferred_element_type=jnp.float32)


_tc_mm = pl.pallas_call(
    _tc_mm_body,
    grid=(G,),
    in_specs=[
        pl.BlockSpec((R, D), lambda i: (i, 0)),
        pl.BlockSpec((D, D), lambda i: (0, 0)),
    ],
    out_specs=pl.BlockSpec((R, D), lambda i: (i, 0)),
    out_shape=jax.ShapeDtypeStruct((N, D), jnp.float32),
)


def _tc_scale_body(x_ref, deg_ref, xs_ref):
    dv = lax.rsqrt(deg_ref[...] + 1.0)
    xs_ref[...] = x_ref[...] * dv


_tc_scale = pl.pallas_call(
    _tc_scale_body,
    grid=(G,),
    in_specs=[
        pl.BlockSpec((R, D), lambda i: (i, 0)),
        pl.BlockSpec((R, 1), lambda i: (i, 0)),
    ],
    out_specs=pl.BlockSpec((R, D), lambda i: (i, 0)),
    out_shape=jax.ShapeDtypeStruct((N, D), jnp.float32),
)


def _tc_mid_body(p0_ref, p1_ref, deg_ref, w_ref, b_ref, xs_ref):
    dv = lax.rsqrt(deg_ref[...] + 1.0)
    h = jnp.maximum((p0_ref[0] + p1_ref[0]) * dv + b_ref[...], 0.0)
    x = jnp.dot(h, w_ref[...], preferred_element_type=jnp.float32)
    xs_ref[...] = x * dv


_tc_mid = pl.pallas_call(
    _tc_mid_body,
    grid=(G,),
    in_specs=[
        pl.BlockSpec((1, R, D), lambda i: (0, i, 0)),
        pl.BlockSpec((1, R, D), lambda i: (1, i, 0)),
        pl.BlockSpec((R, 1), lambda i: (i, 0)),
        pl.BlockSpec((D, D), lambda i: (0, 0)),
        pl.BlockSpec((1, D), lambda i: (0, 0)),
    ],
    out_specs=pl.BlockSpec((R, D), lambda i: (i, 0)),
    out_shape=jax.ShapeDtypeStruct((N, D), jnp.float32),
)


def _tc_fin_body(p0_ref, p1_ref, deg_ref, b_ref, out_ref):
    dv = lax.rsqrt(deg_ref[...] + 1.0)
    out_ref[...] = (p0_ref[0] + p1_ref[0]) * dv + b_ref[...]


_tc_fin = pl.pallas_call(
    _tc_fin_body,
    grid=(G,),
    in_specs=[
        pl.BlockSpec((1, R, D), lambda i: (0, i, 0)),
        pl.BlockSpec((1, R, D), lambda i: (1, i, 0)),
        pl.BlockSpec((R, 1), lambda i: (i, 0)),
        pl.BlockSpec((1, D), lambda i: (0, 0)),
    ],
    out_specs=pl.BlockSpec((R, D), lambda i: (i, 0)),
    out_shape=jax.ShapeDtypeStruct((N, D), jnp.float32),
)


@jax.jit
def kernel(z, edge_index, W1, b1, W2, b2):
    er = edge_index.reshape(2 * NP * NW, CH2, K)
    ones = jnp.ones((K,), jnp.float32)
    zeros1 = jnp.zeros((N,), jnp.float32)

    degp = _sc_degree(er, ones, zeros1)
    x1 = _tc_mm(z, W1)
    deg = (degp[:N] + degp[N:]).reshape(N, 1)

    xs1 = _tc_scale(x1, deg)
    p1 = _sc_scatter(er, xs1)
    xs2 = _tc_mid(p1, p1, deg, W2, b1.reshape(1, D))
    p2 = _sc_scatter(er, xs2)
    return _tc_fin(p2, p2, deg, b2.reshape(1, D))


# speedup vs baseline: 1.1344x; 1.1344x over previous
import functools

import jax
import jax.numpy as jnp
from jax import lax
from jax.experimental import pallas as pl
from jax.experimental.pallas import tpu as pltpu
from jax.experimental.pallas import tpu_sc as plsc

N = 10000
E = 320000
D = 128
NC = 2
NS = 16
NW = NC * NS
EPW = E // NW
K = 125
CH = EPW // K
NP = 2
CH2 = CH // NP
RB = 1000
RB2 = N // NS

_sc_mesh = plsc.VectorSubcoreMesh(core_axis_name="c", subcore_axis_name="s")


@functools.partial(
    pl.kernel,
    out_type=jax.ShapeDtypeStruct((NC * N,), jnp.float32),
    mesh=_sc_mesh,
    scratch_types=[
        pltpu.VMEM((CH2, K), jnp.int32),
        pltpu.VMEM((K,), jnp.float32),
        pltpu.VMEM((RB,), jnp.float32),
        pltpu.VMEM_SHARED((N,), jnp.float32),
    ],
)
def _sc_degree(er_hbm, ones_hbm, zeros1_hbm, deg_hbm, dst_v, ones_v, stage_v,
               acc):
    cid = lax.axis_index("c")
    sid = lax.axis_index("s")
    wid = cid * NS + sid

    @pl.when(sid < N // RB)
    def _():
        rs = pl.ds(sid * RB, RB)
        pltpu.sync_copy(zeros1_hbm.at[rs], stage_v)
        pltpu.sync_copy(stage_v, acc.at[rs])

    pltpu.sync_copy(ones_hbm, ones_v)
    plsc.subcore_barrier()

    for p in range(NP):
        pltpu.sync_copy(er_hbm.at[NP * NW + NP * wid + p], dst_v)

        @pl.loop(0, CH2)
        def _(j):
            pltpu.sync_copy(ones_v, acc.at[dst_v.at[j]], add=True)

    plsc.subcore_barrier()

    @pl.when(sid < N // RB)
    def _():
        rs = pl.ds(sid * RB, RB)
        pltpu.sync_copy(acc.at[rs], stage_v)
        pltpu.sync_copy(stage_v, deg_hbm.at[pl.ds(cid * N + sid * RB, RB)])


@functools.partial(
    pl.kernel,
    out_type=jax.ShapeDtypeStruct((NC, N, D), jnp.float32),
    mesh=_sc_mesh,
    scratch_types=[
        pltpu.VMEM((CH2, K), jnp.int32),
        pltpu.VMEM((CH2, K), jnp.int32),
        pltpu.VMEM((K, D), jnp.float32),
        pltpu.VMEM((K, D), jnp.float32),
        pltpu.SemaphoreType.DMA,
        pltpu.SemaphoreType.DMA,
        pltpu.VMEM_SHARED((N, D), jnp.float32),
    ],
)
def _sc_scatter(er_hbm, xs_hbm, zeros_hbm, out_hbm,
                src_v, dst_v, rows0, rows1, sg0, sg1, acc):
    cid = lax.axis_index("c")
    sid = lax.axis_index("s")
    wid = cid * NS + sid

    @pl.when(sid < N // RB)
    def _():
        rs = pl.ds(sid * RB, RB)

        @pl.when(cid == 0)
        def _():
            pltpu.sync_copy(xs_hbm.at[rs], acc.at[rs])

        @pl.when(cid == 1)
        def _():
            pltpu.sync_copy(zeros_hbm.at[rs], acc.at[rs])

    plsc.subcore_barrier()

    rows = (rows0, rows1)
    sg = (sg0, sg1)

    for p in range(NP):
        pltpu.sync_copy(er_hbm.at[NP * wid + p], src_v)
        pltpu.sync_copy(er_hbm.at[NP * NW + NP * wid + p], dst_v)
        pltpu.async_copy(xs_hbm.at[src_v.at[0]], rows[0], sg[0])

        @pl.loop(0, CH2)
        def _(j):
            for b in range(2):
                @pl.when(j % 2 == b)
                def _():
                    @pl.when(j < CH2 - 1)
                    def _():
                        pltpu.async_copy(
                            xs_hbm.at[src_v.at[j + 1]], rows[1 - b], sg[1 - b])

                    pltpu.make_async_copy(
                        xs_hbm.at[src_v.at[j]], rows[b], sg[b]).wait()
                    pltpu.sync_copy(rows[b], acc.at[dst_v.at[j]], add=True)

    plsc.subcore_barrier()

    @pl.when(sid < N // RB)
    def _():
        rs = pl.ds(sid * RB, RB)
        pltpu.sync_copy(acc.at[rs], out_hbm.at[cid, rs])


R = 1000
G = N // R


def _tc_mm_body(z_ref, w_ref, x_ref):
    x_ref[...] = jnp.dot(z_ref[...], w_ref[...],
                         preferred_element_type=jnp.float32)


_tc_mm = pl.pallas_call(
    _tc_mm_body,
    grid=(G,),
    in_specs=[
        pl.BlockSpec((R, D), lambda i: (i, 0)),
        pl.BlockSpec((D, D), lambda i: (0, 0)),
    ],
    out_specs=pl.BlockSpec((R, D), lambda i: (i, 0)),
    out_shape=jax.ShapeDtypeStruct((N, D), jnp.float32),
)


def _tc_scale_body(x_ref, deg_ref, xs_ref):
    dv = lax.rsqrt(deg_ref[...] + 1.0)
    xs_ref[...] = x_ref[...] * dv


_tc_scale = pl.pallas_call(
    _tc_scale_body,
    grid=(G,),
    in_specs=[
        pl.BlockSpec((R, D), lambda i: (i, 0)),
        pl.BlockSpec((R, 1), lambda i: (i, 0)),
    ],
    out_specs=pl.BlockSpec((R, D), lambda i: (i, 0)),
    out_shape=jax.ShapeDtypeStruct((N, D), jnp.float32),
)


def _tc_mid_body(p0_ref, p1_ref, deg_ref, w_ref, b_ref, xs_ref):
    dv = lax.rsqrt(deg_ref[...] + 1.0)
    h = jnp.maximum((p0_ref[0] + p1_ref[0]) * dv + b_ref[...], 0.0)
    x = jnp.dot(h, w_ref[...], preferred_element_type=jnp.float32)
    xs_ref[...] = x * dv


_tc_mid = pl.pallas_call(
    _tc_mid_body,
    grid=(G,),
    in_specs=[
        pl.BlockSpec((1, R, D), lambda i: (0, i, 0)),
        pl.BlockSpec((1, R, D), lambda i: (1, i, 0)),
        pl.BlockSpec((R, 1), lambda i: (i, 0)),
        pl.BlockSpec((D, D), lambda i: (0, 0)),
        pl.BlockSpec((1, D), lambda i: (0, 0)),
    ],
    out_specs=pl.BlockSpec((R, D), lambda i: (i, 0)),
    out_shape=jax.ShapeDtypeStruct((N, D), jnp.float32),
)


def _tc_fin_body(p0_ref, p1_ref, deg_ref, b_ref, out_ref):
    dv = lax.rsqrt(deg_ref[...] + 1.0)
    out_ref[...] = (p0_ref[0] + p1_ref[0]) * dv + b_ref[...]


_tc_fin = pl.pallas_call(
    _tc_fin_body,
    grid=(G,),
    in_specs=[
        pl.BlockSpec((1, R, D), lambda i: (0, i, 0)),
        pl.BlockSpec((1, R, D), lambda i: (1, i, 0)),
        pl.BlockSpec((R, 1), lambda i: (i, 0)),
        pl.BlockSpec((1, D), lambda i: (0, 0)),
    ],
    out_specs=pl.BlockSpec((R, D), lambda i: (i, 0)),
    out_shape=jax.ShapeDtypeStruct((N, D), jnp.float32),
)


@jax.jit
def kernel(z, edge_index, W1, b1, W2, b2):
    er = edge_index.reshape(2 * NP * NW, CH2, K)
    ones = jnp.ones((K,), jnp.float32)
    zeros1 = jnp.zeros((N,), jnp.float32)
    zeros2 = jnp.zeros((N, D), jnp.float32)

    degp = _sc_degree(er, ones, zeros1)
    x1 = _tc_mm(z, W1)
    deg = (degp[:N] + degp[N:]).reshape(N, 1)

    xs1 = _tc_scale(x1, deg)
    p1 = _sc_scatter(er, xs1, zeros2)
    xs2 = _tc_mid(p1, p1, deg, W2, b1.reshape(1, D))
    p2 = _sc_scatter(er, xs2, zeros2)
    return _tc_fin(p2, p2, deg, b2.reshape(1, D))


# speedup vs baseline: 1.1571x; 1.0201x over previous
import functools

import jax
import jax.numpy as jnp
from jax import lax
from jax.experimental import pallas as pl
from jax.experimental.pallas import tpu as pltpu
from jax.experimental.pallas import tpu_sc as plsc

N = 10000
E = 320000
D = 128
NC = 2
NS = 16
NW = NC * NS
EPW = E // NW
K = 125
CH = EPW // K
NP = 2
CH2 = CH // NP
RB = 1000
RB2 = N // NS

_sc_mesh = plsc.VectorSubcoreMesh(core_axis_name="c", subcore_axis_name="s")


@functools.partial(
    pl.kernel,
    out_type=jax.ShapeDtypeStruct((NC * N,), jnp.float32),
    mesh=_sc_mesh,
    scratch_types=[
        pltpu.VMEM((CH2, K), jnp.int32),
        pltpu.VMEM((K,), jnp.float32),
        pltpu.VMEM((RB,), jnp.float32),
        pltpu.VMEM_SHARED((N,), jnp.float32),
    ],
)
def _sc_degree(er_hbm, ones_hbm, zeros1_hbm, deg_hbm, dst_v, ones_v, stage_v,
               acc):
    cid = lax.axis_index("c")
    sid = lax.axis_index("s")
    wid = cid * NS + sid

    @pl.when(sid < N // RB)
    def _():
        rs = pl.ds(sid * RB, RB)
        pltpu.sync_copy(zeros1_hbm.at[rs], stage_v)
        pltpu.sync_copy(stage_v, acc.at[rs])

    pltpu.sync_copy(ones_hbm, ones_v)
    plsc.subcore_barrier()

    for p in range(NP):
        pltpu.sync_copy(er_hbm.at[NP * NW + NP * wid + p], dst_v)

        @pl.loop(0, CH2)
        def _(j):
            pltpu.sync_copy(ones_v, acc.at[dst_v.at[j]], add=True)

    plsc.subcore_barrier()

    @pl.when(sid < N // RB)
    def _():
        rs = pl.ds(sid * RB, RB)
        pltpu.sync_copy(acc.at[rs], stage_v)
        pltpu.sync_copy(stage_v, deg_hbm.at[pl.ds(cid * N + sid * RB, RB)])


@functools.partial(
    pl.kernel,
    out_type=jax.ShapeDtypeStruct((NC, N, D), jnp.float32),
    mesh=_sc_mesh,
    scratch_types=[
        pltpu.VMEM((CH2, K), jnp.int32),
        pltpu.VMEM((CH2, K), jnp.int32),
        pltpu.VMEM((K, D), jnp.float32),
        pltpu.VMEM((K, D), jnp.float32),
        pltpu.SemaphoreType.DMA,
        pltpu.SemaphoreType.DMA,
        pltpu.VMEM_SHARED((N, D), jnp.float32),
    ],
)
def _sc_scatter(er_hbm, xs_hbm, zeros_hbm, out_hbm,
                src_v, dst_v, rows0, rows1, sg0, sg1, acc):
    cid = lax.axis_index("c")
    sid = lax.axis_index("s")
    wid = cid * NS + sid

    @pl.when(sid < N // RB)
    def _():
        rs = pl.ds(sid * RB, RB)

        @pl.when(cid == 0)
        def _():
            pltpu.sync_copy(xs_hbm.at[rs], acc.at[rs])

        @pl.when(cid == 1)
        def _():
            pltpu.sync_copy(zeros_hbm.at[rs], acc.at[rs])

    plsc.subcore_barrier()

    rows = (rows0, rows1)
    sg = (sg0, sg1)

    for p in range(NP):
        pltpu.sync_copy(er_hbm.at[NP * wid + p], src_v)
        pltpu.sync_copy(er_hbm.at[NP * NW + NP * wid + p], dst_v)
        pltpu.async_copy(xs_hbm.at[src_v.at[0]], rows[0], sg[0])

        @pl.loop(0, CH2)
        def _(j):
            for b in range(2):
                @pl.when(j % 2 == b)
                def _():
                    @pl.when(j < CH2 - 1)
                    def _():
                        pltpu.async_copy(
                            xs_hbm.at[src_v.at[j + 1]], rows[1 - b], sg[1 - b])

                    pltpu.make_async_copy(
                        xs_hbm.at[src_v.at[j]], rows[b], sg[b]).wait()
                    pltpu.sync_copy(rows[b], acc.at[dst_v.at[j]], add=True)

    plsc.subcore_barrier()

    @pl.when(sid < N // RB)
    def _():
        rs = pl.ds(sid * RB, RB)
        pltpu.sync_copy(acc.at[rs], out_hbm.at[cid, rs])


R = 2000
G = N // R


def _tc_mm_body(z_ref, w_ref, x_ref):
    x_ref[...] = jnp.dot(z_ref[...], w_ref[...],
                         preferred_element_type=jnp.float32)


_tc_mm = pl.pallas_call(
    _tc_mm_body,
    grid=(G,),
    in_specs=[
        pl.BlockSpec((R, D), lambda i: (i, 0)),
        pl.BlockSpec((D, D), lambda i: (0, 0)),
    ],
    out_specs=pl.BlockSpec((R, D), lambda i: (i, 0)),
    out_shape=jax.ShapeDtypeStruct((N, D), jnp.float32),
)


def _tc_scale_body(x_ref, deg_ref, xs_ref):
    dv = lax.rsqrt(deg_ref[...] + 1.0)
    xs_ref[...] = x_ref[...] * dv


_tc_scale = pl.pallas_call(
    _tc_scale_body,
    grid=(G,),
    in_specs=[
        pl.BlockSpec((R, D), lambda i: (i, 0)),
        pl.BlockSpec((R, 1), lambda i: (i, 0)),
    ],
    out_specs=pl.BlockSpec((R, D), lambda i: (i, 0)),
    out_shape=jax.ShapeDtypeStruct((N, D), jnp.float32),
)


def _tc_mid_body(p0_ref, p1_ref, deg_ref, w_ref, b_ref, xs_ref):
    dv = lax.rsqrt(deg_ref[...] + 1.0)
    h = jnp.maximum((p0_ref[0] + p1_ref[0]) * dv + b_ref[...], 0.0)
    x = jnp.dot(h, w_ref[...], preferred_element_type=jnp.float32)
    xs_ref[...] = x * dv


_tc_mid = pl.pallas_call(
    _tc_mid_body,
    grid=(G,),
    in_specs=[
        pl.BlockSpec((1, R, D), lambda i: (0, i, 0)),
        pl.BlockSpec((1, R, D), lambda i: (1, i, 0)),
        pl.BlockSpec((R, 1), lambda i: (i, 0)),
        pl.BlockSpec((D, D), lambda i: (0, 0)),
        pl.BlockSpec((1, D), lambda i: (0, 0)),
    ],
    out_specs=pl.BlockSpec((R, D), lambda i: (i, 0)),
    out_shape=jax.ShapeDtypeStruct((N, D), jnp.float32),
)


def _tc_fin_body(p0_ref, p1_ref, deg_ref, b_ref, out_ref):
    dv = lax.rsqrt(deg_ref[...] + 1.0)
    out_ref[...] = (p0_ref[0] + p1_ref[0]) * dv + b_ref[...]


_tc_fin = pl.pallas_call(
    _tc_fin_body,
    grid=(G,),
    in_specs=[
        pl.BlockSpec((1, R, D), lambda i: (0, i, 0)),
        pl.BlockSpec((1, R, D), lambda i: (1, i, 0)),
        pl.BlockSpec((R, 1), lambda i: (i, 0)),
        pl.BlockSpec((1, D), lambda i: (0, 0)),
    ],
    out_specs=pl.BlockSpec((R, D), lambda i: (i, 0)),
    out_shape=jax.ShapeDtypeStruct((N, D), jnp.float32),
)


@jax.jit
def kernel(z, edge_index, W1, b1, W2, b2):
    er = edge_index.reshape(2 * NP * NW, CH2, K)
    ones = jnp.ones((K,), jnp.float32)
    zeros1 = jnp.zeros((N,), jnp.float32)
    zeros2 = jnp.zeros((N, D), jnp.float32)

    degp = _sc_degree(er, ones, zeros1)
    x1 = _tc_mm(z, W1)
    deg = (degp[:N] + degp[N:]).reshape(N, 1)

    xs1 = _tc_scale(x1, deg)
    p1 = _sc_scatter(er, xs1, zeros2)
    xs2 = _tc_mid(p1, p1, deg, W2, b1.reshape(1, D))
    p2 = _sc_scatter(er, xs2, zeros2)
    return _tc_fin(p2, p2, deg, b2.reshape(1, D))
